# Initial kernel scaffold; baseline (speedup 1.0000x reference)
import functools

import jax
import jax.numpy as jnp
from jax import lax
from jax.experimental import pallas as pl
from jax.experimental.pallas import tpu as pltpu
from jax.experimental.pallas import tpu_sc as plsc

_N = 10000
_E = 320000
_D = 128
_H = 128
_DE = 16
_INV_NORM = 1.0 / 100.0

_NC = 2
_NS = 16
_NW = _NC * _NS

_C = 128
_NCHUNKS = _E // _C
_BASE = _NCHUNKS // _NW
_REM = _NCHUNKS - _BASE * _NW
_KMAX = _BASE + 1

_NACC = 10016
_RPT = _NACC // _NS

_mesh = plsc.VectorSubcoreMesh(
    core_axis_name="c", subcore_axis_name="s", num_cores=_NC, num_subcores=_NS
)


def _tile_span():
    c = lax.axis_index("c")
    s = lax.axis_index("s")
    wid = s * _NC + c
    start = wid * _BASE + jnp.minimum(wid, _REM)
    cnt = _BASE + jnp.where(wid < _REM, 1, 0)
    return c, s, start, cnt


def _precompute_body(h_ref, w1_ref, a_ref, b_ref):
    hh = h_ref[...]
    w = w1_ref[...]
    a_ref[...] = jnp.dot(hh, w[0:_D, :], preferred_element_type=jnp.float32)
    b_ref[...] = jnp.dot(hh, w[_D:2 * _D, :], preferred_element_type=jnp.float32)


def _precompute(h, W1):
    bn = 1000
    return pl.pallas_call(
        _precompute_body,
        grid=(_N // bn,),
        in_specs=[
            pl.BlockSpec((bn, _D), lambda i: (i, 0)),
            pl.BlockSpec((2 * _D + _DE, _H), lambda i: (0, 0)),
        ],
        out_specs=[
            pl.BlockSpec((bn, _H), lambda i: (i, 0)),
            pl.BlockSpec((bn, _H), lambda i: (i, 0)),
        ],
        out_shape=[
            jax.ShapeDtypeStruct((_N, _H), jnp.float32),
            jax.ShapeDtypeStruct((_N, _H), jnp.float32),
        ],
    )(h, W1)


@functools.partial(
    pl.kernel,
    out_type=[
        jax.ShapeDtypeStruct((_E, _H), jnp.float32),
        jax.ShapeDtypeStruct((_E, _H), jnp.float32),
    ],
    mesh=_mesh,
    scratch_types=[
        pltpu.VMEM((_KMAX, _C), jnp.int32),
        pltpu.VMEM((_KMAX, _C), jnp.int32),
        pltpu.VMEM((_C, _H), jnp.float32),
        pltpu.VMEM((_C, _H), jnp.float32),
        pltpu.SemaphoreType.DMA,
        pltpu.SemaphoreType.DMA,
    ],
)
def _gather(hA, hB, rows, cols, SA, SB, idxA, idxB, bufA, bufB, semA, semB):
    _, _, start, cnt = _tile_span()
    pltpu.sync_copy(rows.at[pl.ds(start, _KMAX)], idxA)
    pltpu.sync_copy(cols.at[pl.ds(start, _KMAX)], idxB)

    def body(j, carry):
        cpA = pltpu.async_copy(hA.at[idxA.at[j]], bufA, semA)
        cpB = pltpu.async_copy(hB.at[idxB.at[j]], bufB, semB)
        cpA.wait()
        cpB.wait()
        pltpu.sync_copy(bufA, SA.at[pl.ds((start + j) * _C, _C)])
        pltpu.sync_copy(bufB, SB.at[pl.ds((start + j) * _C, _C)])
        return carry

    lax.fori_loop(0, cnt, body, 0)


def _edge_body(sa_ref, sb_ref, ea_ref, w1_ref, b1_ref, w2_ref, b2_ref, o_ref):
    w1c = w1_ref[2 * _D:2 * _D + _DE, :]
    t = (
        sa_ref[...]
        + sb_ref[...]
        + jnp.dot(ea_ref[...], w1c, preferred_element_type=jnp.float32)
        + b1_ref[...]
    )
    t = t * jax.nn.sigmoid(t)
    m = jnp.dot(t, w2_ref[...], preferred_element_type=jnp.float32) + b2_ref[...]
    o_ref[...] = m * jax.nn.sigmoid(m)


def _edge_mlp(SA, SB, edge_attr, W1, b1, W2, b2):
    bn = 1000
    return pl.pallas_call(
        _edge_body,
        grid=(_E // bn,),
        in_specs=[
            pl.BlockSpec((bn, _H), lambda i: (i, 0)),
            pl.BlockSpec((bn, _H), lambda i: (i, 0)),
            pl.BlockSpec((bn, _DE), lambda i: (i, 0)),
            pl.BlockSpec((2 * _D + _DE, _H), lambda i: (0, 0)),
            pl.BlockSpec((1, _H), lambda i: (0, 0)),
            pl.BlockSpec((_H, _H), lambda i: (0, 0)),
            pl.BlockSpec((1, _H), lambda i: (0, 0)),
        ],
        out_specs=pl.BlockSpec((bn, _H), lambda i: (i, 0)),
        out_shape=jax.ShapeDtypeStruct((_E, _H), jnp.float32),
    )(SA, SB, edge_attr, W1, b1, W2, b2)


@functools.partial(
    pl.kernel,
    out_type=jax.ShapeDtypeStruct((_NC, _NACC, _H), jnp.float32),
    mesh=_mesh,
    scratch_types=[
        pltpu.VMEM((_KMAX, _C), jnp.int32),
        pltpu.VMEM((_C, _H), jnp.float32),
        pltpu.VMEM_SHARED((_NACC, _H), jnp.float32),
        pltpu.SemaphoreType.DMA,
    ],
)
def _scatter(mij, rows, zeros, out, idx, buf, acc, sem):
    c, s, start, cnt = _tile_span()
    pltpu.sync_copy(zeros, acc.at[pl.ds(s * _RPT, _RPT)])
    pltpu.sync_copy(rows.at[pl.ds(start, _KMAX)], idx)
    plsc.subcore_barrier()

    def body(j, carry):
        pltpu.sync_copy(mij.at[pl.ds((start + j) * _C, _C)], buf)
        pltpu.sync_copy(buf, acc.at[idx.at[j]], add=True)
        return carry

    lax.fori_loop(0, cnt, body, 0)
    plsc.subcore_barrier()
    pltpu.sync_copy(
        acc.at[pl.ds(s * _RPT, _RPT)], out.at[c, pl.ds(s * _RPT, _RPT)]
    )


def _node_body(h_ref, pp_ref, w3_ref, b3_ref, w4_ref, b4_ref, o_ref):
    p = pp_ref[...]
    agg = (p[0] + p[1]) * _INV_NORM
    hh = h_ref[...]
    x = (
        jnp.dot(hh, w3_ref[0:_D, :], preferred_element_type=jnp.float32)
        + jnp.dot(agg, w3_ref[_D:_D + _H, :], preferred_element_type=jnp.float32)
        + b3_ref[...]
    )
    u = x * jax.nn.sigmoid(x)
    o_ref[...] = (
        hh + jnp.dot(u, w4_ref[...], preferred_element_type=jnp.float32) + b4_ref[...]
    )


def _node_mlp(h, partials, W3, b3, W4, b4):
    bn = 1000
    return pl.pallas_call(
        _node_body,
        grid=(_N // bn,),
        in_specs=[
            pl.BlockSpec((bn, _D), lambda i: (i, 0)),
            pl.BlockSpec((_NC, bn, _H), lambda i: (0, i, 0)),
            pl.BlockSpec((_H + _D, _H), lambda i: (0, 0)),
            pl.BlockSpec((1, _H), lambda i: (0, 0)),
            pl.BlockSpec((_H, _D), lambda i: (0, 0)),
            pl.BlockSpec((1, _D), lambda i: (0, 0)),
        ],
        out_specs=pl.BlockSpec((bn, _D), lambda i: (i, 0)),
        out_shape=jax.ShapeDtypeStruct((_N, _D), jnp.float32),
    )(h, partials, W3, b3, W4, b4)


def kernel(h, edge_index, edge_attr, W1, b1, W2, b2, W3, b3, W4, b4):
    row = edge_index[0].astype(jnp.int32)
    col = edge_index[1].astype(jnp.int32)
    pad = jnp.zeros((_C,), jnp.int32)
    rows2d = jnp.concatenate([row, pad]).reshape(_NCHUNKS + 1, _C)
    cols2d = jnp.concatenate([col, pad]).reshape(_NCHUNKS + 1, _C)

    hA, hB = _precompute(h, W1)
    SA, SB = _gather(hA, hB, rows2d, cols2d)
    mij = _edge_mlp(SA, SB, edge_attr, W1, b1.reshape(1, _H), W2, b2.reshape(1, _H))
    zeros = jnp.zeros((_RPT, _H), jnp.float32)
    partials = _scatter(mij, rows2d, zeros)
    h_out = _node_mlp(h, partials, W3, b3.reshape(1, _H), W4, b4.reshape(1, _D))
    return h_out, mij


# speedup vs baseline: 2.6732x; 2.6732x over previous
import functools

import jax
import jax.numpy as jnp
from jax import lax
from jax.experimental import pallas as pl
from jax.experimental.pallas import tpu as pltpu
from jax.experimental.pallas import tpu_sc as plsc

_N = 10000
_E = 320000
_D = 128
_H = 128
_DE = 16
_INV_NORM = 1.0 / 100.0

_NC = 2
_NS = 16
_NW = _NC * _NS

_C = 128
_NCHUNKS = _E // _C
_BASE = _NCHUNKS // _NW
_REM = _NCHUNKS - _BASE * _NW

_RPT = 632
_NACC = _RPT * _NS

_mesh = plsc.VectorSubcoreMesh(
    core_axis_name="c", subcore_axis_name="s", num_cores=_NC, num_subcores=_NS
)


def _tile_span():
    c = lax.axis_index("c")
    s = lax.axis_index("s")
    wid = s * _NC + c
    start = wid * _BASE + jnp.minimum(wid, _REM)
    cnt = _BASE + jnp.where(wid < _REM, 1, 0)
    return c, s, start, cnt


def _precompute_body(h_ref, w1_ref, a_ref, b_ref):
    hh = h_ref[...]
    w = w1_ref[...]
    a_ref[...] = jnp.dot(hh, w[0:_D, :], preferred_element_type=jnp.float32)
    b_ref[...] = jnp.dot(hh, w[_D:2 * _D, :], preferred_element_type=jnp.float32)


def _precompute(h, W1):
    bn = 1000
    return pl.pallas_call(
        _precompute_body,
        grid=(_N // bn,),
        in_specs=[
            pl.BlockSpec((bn, _D), lambda i: (i, 0)),
            pl.BlockSpec((2 * _D + _DE, _H), lambda i: (0, 0)),
        ],
        out_specs=[
            pl.BlockSpec((bn, _H), lambda i: (i, 0)),
            pl.BlockSpec((bn, _H), lambda i: (i, 0)),
        ],
        out_shape=[
            jax.ShapeDtypeStruct((_N, _H), jnp.float32),
            jax.ShapeDtypeStruct((_N, _H), jnp.float32),
        ],
    )(h, W1)


@functools.partial(
    pl.kernel,
    out_type=[
        jax.ShapeDtypeStruct((_E, _H), jnp.float32),
        jax.ShapeDtypeStruct((_E, _H), jnp.float32),
    ],
    mesh=_mesh,
    scratch_types=[
        pltpu.VMEM((_C,), jnp.int32),
        pltpu.VMEM((_C,), jnp.int32),
        pltpu.VMEM((_C, _H), jnp.float32),
        pltpu.VMEM((_C, _H), jnp.float32),
        pltpu.SemaphoreType.DMA,
        pltpu.SemaphoreType.DMA,
    ],
)
def _gather(hA, hB, rows, cols, SA, SB, idxA, idxB, bufA, bufB, semA, semB):
    _, _, start, cnt = _tile_span()

    def body(j, carry):
        off = (start + j) * _C
        pltpu.sync_copy(rows.at[pl.ds(off, _C)], idxA)
        pltpu.sync_copy(cols.at[pl.ds(off, _C)], idxB)
        cpA = pltpu.async_copy(hA.at[idxA], bufA, semA)
        cpB = pltpu.async_copy(hB.at[idxB], bufB, semB)
        cpA.wait()
        cpB.wait()
        pltpu.sync_copy(bufA, SA.at[pl.ds(off, _C)])
        pltpu.sync_copy(bufB, SB.at[pl.ds(off, _C)])
        return carry

    lax.fori_loop(0, cnt, body, 0)


def _edge_body(sa_ref, sb_ref, ea_ref, w1_ref, b1_ref, w2_ref, b2_ref, o_ref):
    w1c = w1_ref[2 * _D:2 * _D + _DE, :]
    t = (
        sa_ref[...]
        + sb_ref[...]
        + jnp.dot(ea_ref[...], w1c, preferred_element_type=jnp.float32)
        + b1_ref[...]
    )
    t = t * jax.nn.sigmoid(t)
    m = jnp.dot(t, w2_ref[...], preferred_element_type=jnp.float32) + b2_ref[...]
    o_ref[...] = m * jax.nn.sigmoid(m)


def _edge_mlp(SA, SB, edge_attr, W1, b1, W2, b2):
    bn = 1000
    return pl.pallas_call(
        _edge_body,
        grid=(_E // bn,),
        in_specs=[
            pl.BlockSpec((bn, _H), lambda i: (i, 0)),
            pl.BlockSpec((bn, _H), lambda i: (i, 0)),
            pl.BlockSpec((bn, _DE), lambda i: (i, 0)),
            pl.BlockSpec((2 * _D + _DE, _H), lambda i: (0, 0)),
            pl.BlockSpec((1, _H), lambda i: (0, 0)),
            pl.BlockSpec((_H, _H), lambda i: (0, 0)),
            pl.BlockSpec((1, _H), lambda i: (0, 0)),
        ],
        out_specs=pl.BlockSpec((bn, _H), lambda i: (i, 0)),
        out_shape=jax.ShapeDtypeStruct((_E, _H), jnp.float32),
    )(SA, SB, edge_attr, W1, b1, W2, b2)


@functools.partial(
    pl.kernel,
    out_type=jax.ShapeDtypeStruct((_NC, _NACC, _H), jnp.float32),
    mesh=_mesh,
    scratch_types=[
        pltpu.VMEM((_C,), jnp.int32),
        pltpu.VMEM((_C, _H), jnp.float32),
        pltpu.VMEM_SHARED((_NACC, _H), jnp.float32),
    ],
)
def _scatter(mij, rows, zeros, out, idx, buf, acc):
    c, s, start, cnt = _tile_span()
    pltpu.sync_copy(zeros, acc.at[pl.ds(s * _RPT, _RPT)])
    plsc.subcore_barrier()

    def body(j, carry):
        off = (start + j) * _C
        pltpu.sync_copy(rows.at[pl.ds(off, _C)], idx)
        pltpu.sync_copy(mij.at[pl.ds(off, _C)], buf)
        pltpu.sync_copy(buf, acc.at[idx], add=True)
        return carry

    lax.fori_loop(0, cnt, body, 0)
    plsc.subcore_barrier()
    pltpu.sync_copy(
        acc.at[pl.ds(s * _RPT, _RPT)], out.at[c, pl.ds(s * _RPT, _RPT)]
    )


def _node_body(h_ref, pp_ref, w3_ref, b3_ref, w4_ref, b4_ref, o_ref):
    p = pp_ref[...]
    agg = (p[0] + p[1]) * _INV_NORM
    hh = h_ref[...]
    x = (
        jnp.dot(hh, w3_ref[0:_D, :], preferred_element_type=jnp.float32)
        + jnp.dot(agg, w3_ref[_D:_D + _H, :], preferred_element_type=jnp.float32)
        + b3_ref[...]
    )
    u = x * jax.nn.sigmoid(x)
    o_ref[...] = (
        hh + jnp.dot(u, w4_ref[...], preferred_element_type=jnp.float32) + b4_ref[...]
    )


def _node_mlp(h, partials, W3, b3, W4, b4):
    bn = 1000
    return pl.pallas_call(
        _node_body,
        grid=(_N // bn,),
        in_specs=[
            pl.BlockSpec((bn, _D), lambda i: (i, 0)),
            pl.BlockSpec((_NC, bn, _H), lambda i: (0, i, 0)),
            pl.BlockSpec((_H + _D, _H), lambda i: (0, 0)),
            pl.BlockSpec((1, _H), lambda i: (0, 0)),
            pl.BlockSpec((_H, _D), lambda i: (0, 0)),
            pl.BlockSpec((1, _D), lambda i: (0, 0)),
        ],
        out_specs=pl.BlockSpec((bn, _D), lambda i: (i, 0)),
        out_shape=jax.ShapeDtypeStruct((_N, _D), jnp.float32),
    )(h, partials, W3, b3, W4, b4)


def kernel(h, edge_index, edge_attr, W1, b1, W2, b2, W3, b3, W4, b4):
    row = edge_index[0].astype(jnp.int32)
    col = edge_index[1].astype(jnp.int32)

    hA, hB = _precompute(h, W1)
    SA, SB = _gather(hA, hB, row, col)
    mij = _edge_mlp(SA, SB, edge_attr, W1, b1.reshape(1, _H), W2, b2.reshape(1, _H))
    zeros = jnp.zeros((_RPT, _H), jnp.float32)
    partials = _scatter(mij, row, zeros)
    h_out = _node_mlp(h, partials, W3, b3.reshape(1, _H), W4, b4.reshape(1, _D))
    return h_out, mij


# speedup vs baseline: 2.9030x; 1.0860x over previous
import functools

import jax
import jax.numpy as jnp
from jax import lax
from jax.experimental import pallas as pl
from jax.experimental.pallas import tpu as pltpu
from jax.experimental.pallas import tpu_sc as plsc

_N = 10000
_E = 320000
_D = 128
_H = 128
_DE = 16
_INV_NORM = 1.0 / 100.0

_NC = 2
_NS = 16
_NW = _NC * _NS

_C = 128
_NCHUNKS = _E // _C
_BASE = _NCHUNKS // _NW
_REM = _NCHUNKS - _BASE * _NW

_RPT = 632
_NACC = _RPT * _NS

_mesh = plsc.VectorSubcoreMesh(
    core_axis_name="c", subcore_axis_name="s", num_cores=_NC, num_subcores=_NS
)


def _tile_span():
    c = lax.axis_index("c")
    s = lax.axis_index("s")
    wid = s * _NC + c
    start = wid * _BASE + jnp.minimum(wid, _REM)
    cnt = _BASE + jnp.where(wid < _REM, 1, 0)
    return c, s, start, cnt


def _precompute_body(h_ref, w1_ref, a_ref, b_ref):
    hh = h_ref[...]
    w = w1_ref[...]
    a_ref[...] = jnp.dot(hh, w[0:_D, :], preferred_element_type=jnp.float32)
    b_ref[...] = jnp.dot(hh, w[_D:2 * _D, :], preferred_element_type=jnp.float32)


def _precompute(h, W1):
    bn = 1000
    return pl.pallas_call(
        _precompute_body,
        grid=(_N // bn,),
        in_specs=[
            pl.BlockSpec((bn, _D), lambda i: (i, 0)),
            pl.BlockSpec((2 * _D + _DE, _H), lambda i: (0, 0)),
        ],
        out_specs=[
            pl.BlockSpec((bn, _H), lambda i: (i, 0)),
            pl.BlockSpec((bn, _H), lambda i: (i, 0)),
        ],
        out_shape=[
            jax.ShapeDtypeStruct((_N, _H), jnp.float32),
            jax.ShapeDtypeStruct((_N, _H), jnp.float32),
        ],
    )(h, W1)


_KSTAGE = 80


def _gather_span():
    c = lax.axis_index("c")
    s = lax.axis_index("s")
    wid = s * _NC + c
    start = wid * _BASE + 2 * jnp.maximum(wid - (_NW - 2), 0)
    cnt = _BASE + jnp.where(wid >= _NW - 2, 2, 0)
    return start, cnt


@functools.partial(
    pl.kernel,
    out_type=[
        jax.ShapeDtypeStruct((_E, _H), jnp.float32),
        jax.ShapeDtypeStruct((_E, _H), jnp.float32),
    ],
    mesh=_mesh,
    scratch_types=[
        pltpu.VMEM((_KSTAGE * _C,), jnp.int32),
        pltpu.VMEM((_KSTAGE * _C,), jnp.int32),
        pltpu.VMEM((_C, _H), jnp.float32),
        pltpu.VMEM((_C, _H), jnp.float32),
        pltpu.VMEM((_C, _H), jnp.float32),
        pltpu.VMEM((_C, _H), jnp.float32),
        pltpu.SemaphoreType.DMA,
        pltpu.SemaphoreType.DMA,
        pltpu.SemaphoreType.DMA,
        pltpu.SemaphoreType.DMA,
    ],
)
def _gather(hA, hB, rows, cols, SA, SB, idxR, idxC, bA0, bB0, bA1, bB1,
            g0, g1, w0, w1):
    start, cnt = _gather_span()
    pltpu.sync_copy(rows.at[pl.ds(start * _C, _KSTAGE * _C)], idxR)
    pltpu.sync_copy(cols.at[pl.ds(start * _C, _KSTAGE * _C)], idxC)
    slots = ((bA0, bB0, g0, w0), (bA1, bB1, g1, w1))

    for s_ in (0, 1):
        bA, bB, g, _ = slots[s_]
        pltpu.async_copy(hA.at[idxR.at[pl.ds(s_ * _C, _C)]], bA, g)
        pltpu.async_copy(hB.at[idxC.at[pl.ds(s_ * _C, _C)]], bB, g)

    dummy = SA.at[pl.ds(0, _C)]

    def body(p, carry):
        for s_ in (0, 1):
            bA, bB, g, w = slots[s_]
            k = 2 * p + s_
            off = (start + k) * _C
            pltpu.make_async_copy(dummy, bA, g).wait()
            pltpu.make_async_copy(dummy, bB, g).wait()
            pltpu.async_copy(bA, SA.at[pl.ds(off, _C)], w)
            pltpu.async_copy(bB, SB.at[pl.ds(off, _C)], w)

            @pl.when(k + 2 < cnt)
            def _():
                pltpu.make_async_copy(dummy, bA, w).wait()
                pltpu.make_async_copy(dummy, bB, w).wait()
                pltpu.async_copy(hA.at[idxR.at[pl.ds((k + 2) * _C, _C)]], bA, g)
                pltpu.async_copy(hB.at[idxC.at[pl.ds((k + 2) * _C, _C)]], bB, g)

        return carry

    lax.fori_loop(0, cnt // 2, body, 0)
    for s_ in (0, 1):
        bA, bB, _, w = slots[s_]
        pltpu.make_async_copy(dummy, bA, w).wait()
        pltpu.make_async_copy(dummy, bB, w).wait()


def _edge_body(sa_ref, sb_ref, ea_ref, w1_ref, b1_ref, w2_ref, b2_ref, o_ref):
    w1c = w1_ref[2 * _D:2 * _D + _DE, :]
    t = (
        sa_ref[...]
        + sb_ref[...]
        + jnp.dot(ea_ref[...], w1c, preferred_element_type=jnp.float32)
        + b1_ref[...]
    )
    t = t * jax.nn.sigmoid(t)
    m = jnp.dot(t, w2_ref[...], preferred_element_type=jnp.float32) + b2_ref[...]
    o_ref[...] = m * jax.nn.sigmoid(m)


def _edge_mlp(SA, SB, edge_attr, W1, b1, W2, b2):
    bn = 1000
    return pl.pallas_call(
        _edge_body,
        grid=(_E // bn,),
        in_specs=[
            pl.BlockSpec((bn, _H), lambda i: (i, 0)),
            pl.BlockSpec((bn, _H), lambda i: (i, 0)),
            pl.BlockSpec((bn, _DE), lambda i: (i, 0)),
            pl.BlockSpec((2 * _D + _DE, _H), lambda i: (0, 0)),
            pl.BlockSpec((1, _H), lambda i: (0, 0)),
            pl.BlockSpec((_H, _H), lambda i: (0, 0)),
            pl.BlockSpec((1, _H), lambda i: (0, 0)),
        ],
        out_specs=pl.BlockSpec((bn, _H), lambda i: (i, 0)),
        out_shape=jax.ShapeDtypeStruct((_E, _H), jnp.float32),
    )(SA, SB, edge_attr, W1, b1, W2, b2)


@functools.partial(
    pl.kernel,
    out_type=jax.ShapeDtypeStruct((_NC, _NACC, _H), jnp.float32),
    mesh=_mesh,
    scratch_types=[
        pltpu.VMEM((_C,), jnp.int32),
        pltpu.VMEM((_C, _H), jnp.float32),
        pltpu.VMEM_SHARED((_NACC, _H), jnp.float32),
    ],
)
def _scatter(mij, rows, zeros, out, idx, buf, acc):
    c, s, start, cnt = _tile_span()
    pltpu.sync_copy(zeros, acc.at[pl.ds(s * _RPT, _RPT)])
    plsc.subcore_barrier()

    def body(j, carry):
        off = (start + j) * _C
        pltpu.sync_copy(rows.at[pl.ds(off, _C)], idx)
        pltpu.sync_copy(mij.at[pl.ds(off, _C)], buf)
        pltpu.sync_copy(buf, acc.at[idx], add=True)
        return carry

    lax.fori_loop(0, cnt, body, 0)
    plsc.subcore_barrier()
    pltpu.sync_copy(
        acc.at[pl.ds(s * _RPT, _RPT)], out.at[c, pl.ds(s * _RPT, _RPT)]
    )


def _node_body(h_ref, pp_ref, w3_ref, b3_ref, w4_ref, b4_ref, o_ref):
    p = pp_ref[...]
    agg = (p[0] + p[1]) * _INV_NORM
    hh = h_ref[...]
    x = (
        jnp.dot(hh, w3_ref[0:_D, :], preferred_element_type=jnp.float32)
        + jnp.dot(agg, w3_ref[_D:_D + _H, :], preferred_element_type=jnp.float32)
        + b3_ref[...]
    )
    u = x * jax.nn.sigmoid(x)
    o_ref[...] = (
        hh + jnp.dot(u, w4_ref[...], preferred_element_type=jnp.float32) + b4_ref[...]
    )


def _node_mlp(h, partials, W3, b3, W4, b4):
    bn = 1000
    return pl.pallas_call(
        _node_body,
        grid=(_N // bn,),
        in_specs=[
            pl.BlockSpec((bn, _D), lambda i: (i, 0)),
            pl.BlockSpec((_NC, bn, _H), lambda i: (0, i, 0)),
            pl.BlockSpec((_H + _D, _H), lambda i: (0, 0)),
            pl.BlockSpec((1, _H), lambda i: (0, 0)),
            pl.BlockSpec((_H, _D), lambda i: (0, 0)),
            pl.BlockSpec((1, _D), lambda i: (0, 0)),
        ],
        out_specs=pl.BlockSpec((bn, _D), lambda i: (i, 0)),
        out_shape=jax.ShapeDtypeStruct((_N, _D), jnp.float32),
    )(h, partials, W3, b3, W4, b4)


def kernel(h, edge_index, edge_attr, W1, b1, W2, b2, W3, b3, W4, b4):
    row = edge_index[0].astype(jnp.int32)
    col = edge_index[1].astype(jnp.int32)

    hA, hB = _precompute(h, W1)
    SA, SB = _gather(hA, hB, row, col)
    mij = _edge_mlp(SA, SB, edge_attr, W1, b1.reshape(1, _H), W2, b2.reshape(1, _H))
    zeros = jnp.zeros((_RPT, _H), jnp.float32)
    partials = _scatter(mij, row, zeros)
    h_out = _node_mlp(h, partials, W3, b3.reshape(1, _H), W4, b4.reshape(1, _D))
    return h_out, mij


# speedup vs baseline: 3.2838x; 1.1312x over previous
import functools

import jax
import jax.numpy as jnp
from jax import lax
from jax.experimental import pallas as pl
from jax.experimental.pallas import tpu as pltpu
from jax.experimental.pallas import tpu_sc as plsc

_N = 10000
_E = 320000
_D = 128
_H = 128
_DE = 16
_INV_NORM = 1.0 / 100.0

_NC = 2
_NS = 16
_NW = _NC * _NS

_C = 128
_NCHUNKS = _E // _C
_BASE = _NCHUNKS // _NW
_REM = _NCHUNKS - _BASE * _NW

_RPT = 632
_NACC = _RPT * _NS

_mesh = plsc.VectorSubcoreMesh(
    core_axis_name="c", subcore_axis_name="s", num_cores=_NC, num_subcores=_NS
)


def _precompute_body(h_ref, w1_ref, a_ref, b_ref):
    hh = h_ref[...]
    w = w1_ref[...]
    a_ref[...] = jnp.dot(hh, w[0:_D, :], preferred_element_type=jnp.float32)
    b_ref[...] = jnp.dot(hh, w[_D:2 * _D, :], preferred_element_type=jnp.float32)


def _precompute(h, W1):
    bn = 2000
    return pl.pallas_call(
        _precompute_body,
        grid=(_N // bn,),
        in_specs=[
            pl.BlockSpec((bn, _D), lambda i: (i, 0)),
            pl.BlockSpec((2 * _D + _DE, _H), lambda i: (0, 0)),
        ],
        out_specs=[
            pl.BlockSpec((bn, _H), lambda i: (i, 0)),
            pl.BlockSpec((bn, _H), lambda i: (i, 0)),
        ],
        out_shape=[
            jax.ShapeDtypeStruct((_N, _H), jnp.float32),
            jax.ShapeDtypeStruct((_N, _H), jnp.float32),
        ],
    )(h, W1)


_KSTAGE = 80


def _gather_span():
    c = lax.axis_index("c")
    s = lax.axis_index("s")
    wid = s * _NC + c
    start = wid * _BASE + 2 * jnp.maximum(wid - (_NW - 2), 0)
    cnt = _BASE + jnp.where(wid >= _NW - 2, 2, 0)
    return start, cnt


@functools.partial(
    pl.kernel,
    out_type=[
        jax.ShapeDtypeStruct((_E, _H), jnp.float32),
        jax.ShapeDtypeStruct((_E, _H), jnp.float32),
    ],
    mesh=_mesh,
    scratch_types=[
        pltpu.VMEM((_KSTAGE * _C,), jnp.int32),
        pltpu.VMEM((_KSTAGE * _C,), jnp.int32),
        pltpu.VMEM((_C, _H), jnp.float32),
        pltpu.VMEM((_C, _H), jnp.float32),
        pltpu.VMEM((_C, _H), jnp.float32),
        pltpu.VMEM((_C, _H), jnp.float32),
        pltpu.SemaphoreType.DMA,
        pltpu.SemaphoreType.DMA,
        pltpu.SemaphoreType.DMA,
        pltpu.SemaphoreType.DMA,
    ],
)
def _gather(hA, hB, rows, cols, SA, SB, idxR, idxC, bA0, bB0, bA1, bB1,
            g0, g1, w0, w1):
    start, cnt = _gather_span()
    pltpu.sync_copy(rows.at[pl.ds(start * _C, _KSTAGE * _C)], idxR)
    pltpu.sync_copy(cols.at[pl.ds(start * _C, _KSTAGE * _C)], idxC)
    slots = ((bA0, bB0, g0, w0), (bA1, bB1, g1, w1))

    for s_ in (0, 1):
        bA, bB, g, _ = slots[s_]
        pltpu.async_copy(hA.at[idxR.at[pl.ds(s_ * _C, _C)]], bA, g)
        pltpu.async_copy(hB.at[idxC.at[pl.ds(s_ * _C, _C)]], bB, g)

    dummy = SA.at[pl.ds(0, _C)]

    def body(p, carry):
        for s_ in (0, 1):
            bA, bB, g, w = slots[s_]
            k = 2 * p + s_
            off = (start + k) * _C
            pltpu.make_async_copy(dummy, bA, g).wait()
            pltpu.make_async_copy(dummy, bB, g).wait()
            pltpu.async_copy(bA, SA.at[pl.ds(off, _C)], w)
            pltpu.async_copy(bB, SB.at[pl.ds(off, _C)], w)

            @pl.when(k + 2 < cnt)
            def _():
                pltpu.make_async_copy(dummy, bA, w).wait()
                pltpu.make_async_copy(dummy, bB, w).wait()
                pltpu.async_copy(hA.at[idxR.at[pl.ds((k + 2) * _C, _C)]], bA, g)
                pltpu.async_copy(hB.at[idxC.at[pl.ds((k + 2) * _C, _C)]], bB, g)

        return carry

    lax.fori_loop(0, cnt // 2, body, 0)
    for s_ in (0, 1):
        bA, bB, _, w = slots[s_]
        pltpu.make_async_copy(dummy, bA, w).wait()
        pltpu.make_async_copy(dummy, bB, w).wait()


def _edge_body(sa_ref, sb_ref, ea_ref, w1_ref, b1_ref, w2_ref, b2_ref, o_ref):
    w1c = w1_ref[2 * _D:2 * _D + _DE, :]
    t = (
        sa_ref[...].astype(jnp.float32)
        + sb_ref[...].astype(jnp.float32)
        + jnp.dot(ea_ref[...], w1c, preferred_element_type=jnp.float32)
        + b1_ref[...]
    )
    t = t * jax.nn.sigmoid(t)
    m = (
        jnp.dot(
            t.astype(jnp.bfloat16),
            w2_ref[...].astype(jnp.bfloat16),
            preferred_element_type=jnp.float32,
        )
        + b2_ref[...]
    )
    o_ref[...] = m * jax.nn.sigmoid(m)


def _edge_mlp(SA, SB, edge_attr, W1, b1, W2, b2):
    bn = 1000
    return pl.pallas_call(
        _edge_body,
        grid=(_E // bn,),
        in_specs=[
            pl.BlockSpec((bn, _H), lambda i: (i, 0)),
            pl.BlockSpec((bn, _H), lambda i: (i, 0)),
            pl.BlockSpec((bn, _DE), lambda i: (i, 0)),
            pl.BlockSpec((2 * _D + _DE, _H), lambda i: (0, 0)),
            pl.BlockSpec((1, _H), lambda i: (0, 0)),
            pl.BlockSpec((_H, _H), lambda i: (0, 0)),
            pl.BlockSpec((1, _H), lambda i: (0, 0)),
        ],
        out_specs=pl.BlockSpec((bn, _H), lambda i: (i, 0)),
        out_shape=jax.ShapeDtypeStruct((_E, _H), jnp.float32),
    )(SA, SB, edge_attr, W1, b1, W2, b2)


@functools.partial(
    pl.kernel,
    out_type=jax.ShapeDtypeStruct((_NC, _NACC, _H), jnp.float32),
    mesh=_mesh,
    scratch_types=[
        pltpu.VMEM((_C,), jnp.int32),
        pltpu.VMEM((_C,), jnp.int32),
        pltpu.VMEM((_C, _H), jnp.float32),
        pltpu.VMEM((_C, _H), jnp.float32),
        pltpu.VMEM_SHARED((_NACC, _H), jnp.float32),
        pltpu.SemaphoreType.DMA,
        pltpu.SemaphoreType.DMA,
    ],
)
def _scatter(mij, rows, zeros, out, idx0, idx1, buf0, buf1, acc, m0, m1):
    c = lax.axis_index("c")
    s = lax.axis_index("s")
    start, cnt = _gather_span()
    pltpu.sync_copy(zeros, acc.at[pl.ds(s * _RPT, _RPT)])
    plsc.subcore_barrier()

    slots = ((idx0, buf0, m0), (idx1, buf1, m1))
    for s_ in (0, 1):
        idx, buf, m = slots[s_]
        off = (start + s_) * _C
        pltpu.async_copy(rows.at[pl.ds(off, _C)], idx, m)
        pltpu.async_copy(mij.at[pl.ds(off, _C)], buf, m)

    didx = rows.at[pl.ds(0, _C)]
    dbuf = mij.at[pl.ds(0, _C)]

    def body(p, carry):
        for s_ in (0, 1):
            idx, buf, m = slots[s_]
            k = 2 * p + s_
            pltpu.make_async_copy(didx, idx, m).wait()
            pltpu.make_async_copy(dbuf, buf, m).wait()
            pltpu.sync_copy(buf, acc.at[idx], add=True)

            @pl.when(k + 2 < cnt)
            def _():
                off = (start + k + 2) * _C
                pltpu.async_copy(rows.at[pl.ds(off, _C)], idx, m)
                pltpu.async_copy(mij.at[pl.ds(off, _C)], buf, m)

        return carry

    lax.fori_loop(0, cnt // 2, body, 0)
    plsc.subcore_barrier()
    pltpu.sync_copy(
        acc.at[pl.ds(s * _RPT, _RPT)], out.at[c, pl.ds(s * _RPT, _RPT)]
    )


def _node_body(h_ref, pp_ref, w3_ref, b3_ref, w4_ref, b4_ref, o_ref):
    p = pp_ref[...]
    agg = (p[0] + p[1]) * _INV_NORM
    hh = h_ref[...]
    x = (
        jnp.dot(hh, w3_ref[0:_D, :], preferred_element_type=jnp.float32)
        + jnp.dot(agg, w3_ref[_D:_D + _H, :], preferred_element_type=jnp.float32)
        + b3_ref[...]
    )
    u = x * jax.nn.sigmoid(x)
    o_ref[...] = (
        hh + jnp.dot(u, w4_ref[...], preferred_element_type=jnp.float32) + b4_ref[...]
    )


def _node_mlp(h, partials, W3, b3, W4, b4):
    bn = 1000
    return pl.pallas_call(
        _node_body,
        grid=(_N // bn,),
        in_specs=[
            pl.BlockSpec((bn, _D), lambda i: (i, 0)),
            pl.BlockSpec((_NC, bn, _H), lambda i: (0, i, 0)),
            pl.BlockSpec((_H + _D, _H), lambda i: (0, 0)),
            pl.BlockSpec((1, _H), lambda i: (0, 0)),
            pl.BlockSpec((_H, _D), lambda i: (0, 0)),
            pl.BlockSpec((1, _D), lambda i: (0, 0)),
        ],
        out_specs=pl.BlockSpec((bn, _D), lambda i: (i, 0)),
        out_shape=jax.ShapeDtypeStruct((_N, _D), jnp.float32),
    )(h, partials, W3, b3, W4, b4)


def kernel(h, edge_index, edge_attr, W1, b1, W2, b2, W3, b3, W4, b4):
    row = edge_index[0].astype(jnp.int32)
    col = edge_index[1].astype(jnp.int32)

    hA, hB = _precompute(h, W1)
    SA, SB = _gather(hA, hB, row, col)
    mij = _edge_mlp(SA, SB, edge_attr, W1, b1.reshape(1, _H), W2, b2.reshape(1, _H))
    zeros = jnp.zeros((_RPT, _H), jnp.float32)
    partials = _scatter(mij, row, zeros)
    h_out = _node_mlp(h, partials, W3, b3.reshape(1, _H), W4, b4.reshape(1, _D))
    return h_out, mij
